# baseline (speedup 1.0000x reference)
import jax
import jax.numpy as jnp
from jax import lax
from jax.experimental import pallas as pl
from jax.experimental.pallas import tpu as pltpu
from jax.experimental.pallas import tpu_sc as plsc

D = 64
KNEG = 20
NC, NS = 2, 16
NW = NC * NS
ROWS = 24


def _sc_scores_body(in_hbm, out_hbm, cen_hbm, ctx_hbm, neg_hbm, scores_hbm,
                    idx_c, idx_o, idx_n, vc_rows, vo_g, vng, scores_v, sem):
    bpw = vc_rows.shape[0]
    ng = bpw // 16
    wid = lax.axis_index("s") * NC + lax.axis_index("c")
    base = wid * bpw

    pltpu.sync_copy(cen_hbm.at[pl.ds(base, bpw)], idx_c)
    pltpu.sync_copy(ctx_hbm.at[pl.ds(base, bpw)], idx_o)
    pltpu.sync_copy(neg_hbm.at[pl.ds(base * KNEG, bpw * KNEG)], idx_n)

    cps = [pltpu.async_copy(in_hbm.at[idx_c.at[pl.ds(c * 128, 128)]],
                            vc_rows.at[pl.ds(c * 128, 128), :], sem)
           for c in range(bpw // 128)]
    for cp in cps:
        cp.wait()

    iota = lax.iota(jnp.int32, 16)
    iota_k = iota * KNEG
    big = jnp.full((16,), 1e4, jnp.float32)

    @pl.loop(0, ng)
    def _(g):
        col0 = pl.multiple_of(g * 16, 16)
        nbase = pl.multiple_of(g * (16 * KNEG), 8)
        cps = [
            pltpu.async_copy(out_hbm.at[idx_o.at[pl.ds(col0, 16)]], vo_g, sem),
            pltpu.async_copy(out_hbm.at[idx_n.at[pl.ds(nbase, 128)]],
                             vng.at[pl.ds(0, 128), :], sem),
            pltpu.async_copy(out_hbm.at[idx_n.at[pl.ds(nbase + 128, 128)]],
                             vng.at[pl.ds(128, 128), :], sem),
            pltpu.async_copy(out_hbm.at[idx_n.at[pl.ds(nbase + 256, 64)]],
                             vng.at[pl.ds(256, 64), :], sem),
        ]
        for cp in cps:
            cp.wait()

        vc_idx = col0 + iota
        accs = [jnp.zeros((16,), jnp.float32)] * (KNEG + 1)
        for d in range(D):
            dcol = jnp.full((16,), d, jnp.int32)
            vcc = plsc.load_gather(vc_rows, [vc_idx, dcol])
            voc = plsc.load_gather(vo_g, [iota, dcol])
            accs[0] = accs[0] + vcc * voc
            for k in range(KNEG):
                vnc = plsc.load_gather(vng, [iota_k + k, dcol])
                accs[k + 1] = accs[k + 1] + vnc * vcc
        scores_v[0, pl.ds(col0, 16)] = accs[0]
        for k in range(KNEG):
            scores_v[k + 1, pl.ds(col0, 16)] = -accs[k + 1]
        for r in range(KNEG + 1, ROWS):
            scores_v[r, pl.ds(col0, 16)] = big

    pltpu.sync_copy(scores_v, scores_hbm.at[:, pl.ds(base, bpw)])


def _tc_loss_body(s_ref, o_ref):
    x = s_ref[...]
    ls = jnp.minimum(x, 0.0) - jnp.log1p(jnp.exp(-jnp.abs(x)))
    row = lax.broadcasted_iota(jnp.int32, x.shape, 0)
    ls = jnp.where(row < KNEG + 1, ls, 0.0)
    o_ref[0, 0] = -jnp.sum(ls) / s_ref.shape[1]


def kernel(center, context, negatives, in_embed, out_embed):
    b = center.shape[0]
    bpw = b // NW
    negflat = negatives.reshape(-1)

    scores = pl.kernel(
        _sc_scores_body,
        out_type=jax.ShapeDtypeStruct((ROWS, b), jnp.float32),
        mesh=plsc.VectorSubcoreMesh(core_axis_name="c", subcore_axis_name="s"),
        compiler_params=pltpu.CompilerParams(
            needs_layout_passes=False, use_tc_tiling_on_sc=False),
        scratch_types=[
            pltpu.VMEM((bpw,), jnp.int32),
            pltpu.VMEM((bpw,), jnp.int32),
            pltpu.VMEM((bpw * KNEG,), jnp.int32),
            pltpu.VMEM((bpw, D), jnp.float32),
            pltpu.VMEM((16, D), jnp.float32),
            pltpu.VMEM((16 * KNEG, D), jnp.float32),
            pltpu.VMEM((ROWS, bpw), jnp.float32),
            pltpu.SemaphoreType.DMA,
        ],
    )(in_embed, out_embed, center, context, negflat)

    loss = pl.pallas_call(
        _tc_loss_body,
        out_shape=jax.ShapeDtypeStruct((1, 1), jnp.float32),
        in_specs=[pl.BlockSpec((ROWS, b), lambda: (0, 0))],
        out_specs=pl.BlockSpec(memory_space=pltpu.SMEM),
    )(scores)
    return loss[0, 0]
